# baseline (speedup 1.0000x reference)
import functools
import jax
import jax.numpy as jnp
from jax import lax
from jax.experimental import pallas as pl
from jax.experimental.pallas import tpu as pltpu
from jax.experimental.pallas import tpu_sc as plsc

N = 10000
E = 160000
D_IN = 128
D_H = 256
K = 512
CC = 0.25
SCE = 2

NTILE = 16
CHUNK = 125
ROWS_PER_TILE = 80
NROW2D = E // CHUNK
NPAD = 10240
NPT = NPAD // NTILE

_mesh = plsc.VectorSubcoreMesh(core_axis_name="c", subcore_axis_name="s")


HROW = 5120
AROW = 5248
JUNK = 5192
CCHUNK = 128
CROWS = 80
EPAD = 16 * CROWS * CCHUNK


def _remap(idx_d, g, p, idx_r):
    for j in range(CCHUNK // 16):
        v = idx_d[g, pl.ds(16 * j, 16)]
        lo = v - jnp.int32(p * HROW)
        ok = (lo >= 0) & (lo < HROW)
        idx_r[pl.ds(16 * j, 16)] = jnp.where(ok, lo, jnp.int32(JUNK))


def _conv_body(h0, h1, src2d, dst2d, zeros2d, out0, out1,
               idx_s, idx_d, idx_r0, idx_r1, rows0, rows1, acc,
               sem0, sem1, sem2, sem3):
    s = lax.axis_index("s")
    c = lax.axis_index("c")

    pltpu.sync_copy(src2d.at[pl.ds(s * CROWS, CROWS)], idx_s)
    pltpu.sync_copy(dst2d.at[pl.ds(s * CROWS, CROWS)], idx_d)

    def row_pass(p, h_hbm, out_hbm):
        pltpu.sync_copy(zeros2d, acc.at[pl.ds(s * (AROW // 16), AROW // 16)])
        plsc.subcore_barrier()

        pltpu.async_copy(h_hbm.at[idx_s.at[0]], rows0, sem0)
        pltpu.async_copy(h_hbm.at[idx_s.at[1]], rows1, sem1)

        def step(i, _):
            g = 2 * i
            pltpu.make_async_copy(h_hbm.at[idx_s.at[g]], rows0, sem0).wait()
            _remap(idx_d, g, p, idx_r0)
            pltpu.async_copy(rows0, acc.at[idx_r0], sem2, add=True)

            pltpu.make_async_copy(h_hbm.at[idx_s.at[g + 1]], rows1, sem1).wait()
            _remap(idx_d, g + 1, p, idx_r1)
            pltpu.async_copy(rows1, acc.at[idx_r1], sem3, add=True)

            pltpu.make_async_copy(rows0, acc.at[idx_r0], sem2).wait()

            @pl.when(g + 2 < CROWS)
            def _():
                pltpu.async_copy(h_hbm.at[idx_s.at[g + 2]], rows0, sem0)

            pltpu.make_async_copy(rows1, acc.at[idx_r1], sem3).wait()

            @pl.when(g + 3 < CROWS)
            def _():
                pltpu.async_copy(h_hbm.at[idx_s.at[g + 3]], rows1, sem1)
            return 0

        lax.fori_loop(0, CROWS // 2, step, 0)
        plsc.subcore_barrier()
        pltpu.sync_copy(acc.at[pl.ds(s * 320, 320)],
                        out_hbm.at[pl.ds(p * HROW + s * 320, 320)])
        plsc.subcore_barrier()

    def col_half(h_hbm, out_hbm):
        row_pass(0, h_hbm, out_hbm)
        row_pass(1, h_hbm, out_hbm)

    @pl.when(c == 0)
    def _():
        col_half(h0, out0)

    @pl.when(c == 1)
    def _():
        col_half(h1, out1)


def _sc_conv(h0, h1, src2d, dst2d, zeros2d):
    f = pl.kernel(
        _conv_body,
        out_type=[jax.ShapeDtypeStruct((NPAD, 128), jnp.float32)] * 2,
        mesh=_mesh,
        scratch_types=[
            pltpu.VMEM((CROWS, CCHUNK), jnp.int32),
            pltpu.VMEM((CROWS, CCHUNK), jnp.int32),
            pltpu.VMEM((CCHUNK,), jnp.int32),
            pltpu.VMEM((CCHUNK,), jnp.int32),
            pltpu.VMEM((CCHUNK, 128), jnp.float32),
            pltpu.VMEM((CCHUNK, 128), jnp.float32),
            pltpu.VMEM_SHARED((AROW, 128), jnp.float32),
            pltpu.SemaphoreType.DMA,
            pltpu.SemaphoreType.DMA,
            pltpu.SemaphoreType.DMA,
            pltpu.SemaphoreType.DMA,
        ],
    )
    return f(h0, h1, src2d, dst2d, zeros2d)


def _deg_body(i0, i1, i2, i3, i4, i5, zeros640, o0, o1, o2, o3, o4, o5,
              idxblk, ones, a0, a1, a2, sem):
    s = lax.axis_index("s")
    c = lax.axis_index("c")

    for j in range(8):
        ones[pl.ds(16 * j, 16)] = jnp.ones((16,), jnp.float32)

    def histo(idx_hbm, acc):
        pltpu.sync_copy(zeros640, acc.at[pl.ds(s * 640, 640)])
        pltpu.sync_copy(idx_hbm.at[pl.ds(s * ROWS_PER_TILE, ROWS_PER_TILE)],
                        idxblk)
        plsc.subcore_barrier()

        def step(i, _):
            for j in range(10):
                pltpu.async_copy(ones.at[pl.ds(0, CHUNK)],
                                 acc.at[idxblk.at[10 * i + j]], sem, add=True)
            for j in range(10):
                pltpu.make_async_copy(ones.at[pl.ds(0, CHUNK)],
                                      acc.at[idxblk.at[10 * i + j]], sem).wait()
            return 0

        lax.fori_loop(0, ROWS_PER_TILE // 10, step, 0)
        plsc.subcore_barrier()

    def emit(idx_hbm, acc, out_hbm):
        histo(idx_hbm, acc)
        pltpu.sync_copy(acc.at[pl.ds(s * 640, 640)],
                        out_hbm.at[pl.ds(s * 640, 640)])

    @pl.when(c == 0)
    def _():
        emit(i0, a0, o0)
        emit(i1, a1, o1)
        emit(i2, a2, o2)

    @pl.when(c == 1)
    def _():
        emit(i3, a0, o3)
        emit(i4, a1, o4)
        emit(i5, a2, o5)


def _sc_degrees(idx6, zeros640):
    f = pl.kernel(
        _deg_body,
        out_type=[jax.ShapeDtypeStruct((NPAD,), jnp.float32)] * 6,
        mesh=_mesh,
        scratch_types=[
            pltpu.VMEM((ROWS_PER_TILE, CHUNK), jnp.int32),
            pltpu.VMEM((128,), jnp.float32),
            pltpu.VMEM_SHARED((NPAD,), jnp.float32),
            pltpu.VMEM_SHARED((NPAD,), jnp.float32),
            pltpu.VMEM_SHARED((NPAD,), jnp.float32),
            pltpu.SemaphoreType.DMA,
        ],
    )
    return f(*idx6, zeros640)


def _vq_body(xn_ref, cn_ref, idx_ref):
    xn = xn_ref[...]
    cn = cn_ref[...]
    dot = lax.dot_general(xn, cn, (((1,), (1,)), ((), ())),
                          preferred_element_type=jnp.float32)
    a = jnp.sum(xn * xn, axis=1, keepdims=True)
    sc = jnp.sum(cn * cn, axis=1)[None, :]
    d = a + sc - 2.0 * dot
    idx_ref[0, 0, :] = jnp.argmin(d, axis=1).astype(jnp.int32)


def _vq_argmin(xn, cn):
    BR = 1000
    idx = pl.pallas_call(
        _vq_body,
        grid=(N // BR,),
        in_specs=[
            pl.BlockSpec((BR, D_H), lambda i: (i, 0)),
            pl.BlockSpec((K, D_H), lambda i: (0, 0)),
        ],
        out_specs=pl.BlockSpec((1, 1, BR), lambda i: (i, 0, 0)),
        out_shape=jax.ShapeDtypeStruct((N // BR, 1, BR), jnp.int32),
    )(xn, cn)
    return idx.reshape(N)


def _normalize(x, eps=1e-12):
    n = jnp.linalg.norm(x, axis=-1, keepdims=True)
    return x / jnp.maximum(n, eps)


def _norm_coeff(deg):
    return jnp.where(deg > 0, 1.0 / jnp.sqrt(jnp.maximum(deg, 1e-9)), 0.0)


def _bn(x, g, b):
    mu = jnp.mean(x, axis=0)
    var = jnp.var(x, axis=0)
    return (x - mu) / jnp.sqrt(var + 1e-5) * g + b


def _hetero_sc(x, conv_params, ed, zeros2d):
    out = 0.0
    for r in ('SEQ', 'KNN', 'DIS'):
        W, b = conv_params[r]['W'], conv_params[r]['b']
        src2d, dst2d, _, _, ns, nd = ed[r]
        h = (x @ W) * ns[:, None]
        a0, a1 = _sc_conv(h[:, :128], h[:, 128:], src2d, dst2d, zeros2d)
        agg = jnp.concatenate([a0[:N], a1[:N]], axis=1)
        out = out + (agg * nd[:, None] + b)
    return out


def _hetero_exact(x, conv_params, ed):
    out = 0.0
    for r in ('SEQ', 'KNN', 'DIS'):
        W, b = conv_params[r]['W'], conv_params[r]['b']
        src, dst, ns, nd = ed[r][2], ed[r][3], ed[r][4], ed[r][5]
        h = x @ W
        m = h[src] * ns[src][:, None]
        agg = jnp.zeros((N, h.shape[1]), h.dtype).at[dst].add(m)
        out = out + (agg * nd[:, None] + b)
    return out


def _encode(x, enc, ed):
    for l in range(2):
        x = _hetero_exact(x, enc['convs'][l], ed)
        x = x @ enc['fcs'][l]['W'] + enc['fcs'][l]['b']
        x = _bn(jax.nn.relu(x), enc['bns'][l]['g'], enc['bns'][l]['b'])
    return x


def _decode(e, dec, ed, zeros2d):
    x = _hetero_sc(e, dec['convs'][0], ed, zeros2d)
    x = x @ dec['fcs'][0]['W'] + dec['fcs'][0]['b']
    x = _bn(jax.nn.relu(x), dec['bns'][0]['g'], dec['bns'][0]['b'])
    x = _hetero_sc(x, dec['convs'][1], ed, zeros2d)
    x = x @ dec['fcs'][1]['W'] + dec['fcs'][1]['b']
    return x


def kernel(x, edge_index_seq, edge_index_knn, edge_index_dis, mask, params,
           codebook):
    maskf = mask.astype(jnp.float32)
    zeros2d = jnp.zeros((AROW // 16, 128), jnp.float32)
    zeros640 = jnp.zeros((640,), jnp.float32)
    pad_src = jnp.arange(EPAD - E, dtype=jnp.int32) % N
    pad_dst = jnp.full((EPAD - E,), 1 << 30, jnp.int32)

    e2d, ec = {}, {}
    for r, ei in (('SEQ', edge_index_seq), ('KNN', edge_index_knn),
                  ('DIS', edge_index_dis)):
        e2d[r] = (ei[0].reshape(NROW2D, CHUNK), ei[1].reshape(NROW2D, CHUNK))
        ec[r] = (jnp.concatenate([ei[0], pad_src]).reshape(EPAD // CCHUNK, CCHUNK),
                 jnp.concatenate([ei[1], pad_dst]).reshape(EPAD // CCHUNK, CCHUNK))

    idx6 = [e2d['SEQ'][0], e2d['SEQ'][1], e2d['KNN'][0],
            e2d['KNN'][1], e2d['DIS'][0], e2d['DIS'][1]]
    degs = _sc_degrees(idx6, zeros640)
    srcdst = {'SEQ': edge_index_seq, 'KNN': edge_index_knn,
              'DIS': edge_index_dis}
    ed = {}
    for k, r in enumerate(('SEQ', 'KNN', 'DIS')):
        ns = _norm_coeff(degs[2 * k][:N])
        nd = _norm_coeff(degs[2 * k + 1][:N])
        ed[r] = (ec[r][0], ec[r][1], srcdst[r][0], srcdst[r][1], ns, nd)

    x_in = x
    z = _encode(x_in, params['enc'], ed)
    xn = _normalize(z)
    cn = _normalize(codebook)
    idx = _vq_argmin(xn, cn)
    onehot = (idx[:, None] == jnp.arange(K)[None, :]).astype(jnp.float32)
    quant = onehot @ cn
    q_loss = jnp.mean((quant - xn) ** 2)
    e_q_loss = q_loss + CC * q_loss
    e = xn + (quant - xn)
    x_recon = _decode(e, params['dec'], ed, zeros2d)
    recon_loss = jnp.mean((x_recon - x_in) ** 2)
    mi = onehot @ maskf
    e_masked = e * (1.0 - mi)[:, None]
    x_mask_recon = _decode(e_masked, params['dec'], ed, zeros2d)
    a = _normalize(x_mask_recon)
    b = _normalize(x_in)
    per_node = (1.0 - jnp.sum(a * b, axis=-1)) ** SCE
    mask_loss = jnp.sum(per_node * mi) / (jnp.sum(mi) + 1e-12)
    return z, e_masked, e_q_loss, recon_loss, mask_loss


# speedup vs baseline: 1.0436x; 1.0436x over previous
import functools
import jax
import jax.numpy as jnp
from jax import lax
from jax.experimental import pallas as pl
from jax.experimental.pallas import tpu as pltpu
from jax.experimental.pallas import tpu_sc as plsc

N = 10000
E = 160000
D_IN = 128
D_H = 256
K = 512
CC = 0.25
SCE = 2

NTILE = 16
CHUNK = 125
ROWS_PER_TILE = 80
NROW2D = E // CHUNK
NPAD = 10240
NPT = NPAD // NTILE

_mesh = plsc.VectorSubcoreMesh(core_axis_name="c", subcore_axis_name="s")


HROW = 5120
AROW = 5248
JUNK = 5192
CCHUNK = 128
CROWS = 80
EPAD = 16 * CROWS * CCHUNK


def _remap(idx_d, g, p, idx_r):
    for j in range(CCHUNK // 16):
        v = idx_d[g, pl.ds(16 * j, 16)]
        lo = v - jnp.int32(p * HROW)
        ok = (lo >= 0) & (lo < HROW)
        idx_r[pl.ds(16 * j, 16)] = jnp.where(ok, lo, jnp.int32(JUNK))


def _conv_body(h0, h1, src2d, dst2d, zeros2d, out0, out1,
               idx_s, idx_d, idx_r0, idx_r1, rows0, rows1, acc,
               sem0, sem1, sem2, sem3):
    s = lax.axis_index("s")
    c = lax.axis_index("c")

    pltpu.sync_copy(src2d.at[pl.ds(s * CROWS, CROWS)], idx_s)
    pltpu.sync_copy(dst2d.at[pl.ds(s * CROWS, CROWS)], idx_d)

    def row_pass(p, h_hbm, out_hbm):
        pltpu.sync_copy(zeros2d, acc.at[pl.ds(s * (AROW // 16), AROW // 16)])
        plsc.subcore_barrier()

        pltpu.async_copy(h_hbm.at[idx_s.at[0]], rows0, sem0)
        pltpu.async_copy(h_hbm.at[idx_s.at[1]], rows1, sem1)

        def step(i, _):
            g = 2 * i
            pltpu.make_async_copy(h_hbm.at[idx_s.at[g]], rows0, sem0).wait()
            _remap(idx_d, g, p, idx_r0)
            pltpu.sync_copy(rows0, acc.at[idx_r0], add=True)

            @pl.when(g + 2 < CROWS)
            def _():
                pltpu.async_copy(h_hbm.at[idx_s.at[g + 2]], rows0, sem0)

            pltpu.make_async_copy(h_hbm.at[idx_s.at[g + 1]], rows1, sem1).wait()
            _remap(idx_d, g + 1, p, idx_r1)
            pltpu.sync_copy(rows1, acc.at[idx_r1], add=True)

            @pl.when(g + 3 < CROWS)
            def _():
                pltpu.async_copy(h_hbm.at[idx_s.at[g + 3]], rows1, sem1)
            return 0

        lax.fori_loop(0, CROWS // 2, step, 0)
        plsc.subcore_barrier()
        pltpu.sync_copy(acc.at[pl.ds(s * 320, 320)],
                        out_hbm.at[pl.ds(p * HROW + s * 320, 320)])
        plsc.subcore_barrier()

    def col_half(h_hbm, out_hbm):
        row_pass(0, h_hbm, out_hbm)
        row_pass(1, h_hbm, out_hbm)

    @pl.when(c == 0)
    def _():
        col_half(h0, out0)

    @pl.when(c == 1)
    def _():
        col_half(h1, out1)


def _sc_conv(h0, h1, src2d, dst2d, zeros2d):
    f = pl.kernel(
        _conv_body,
        out_type=[jax.ShapeDtypeStruct((NPAD, 128), jnp.float32)] * 2,
        mesh=_mesh,
        scratch_types=[
            pltpu.VMEM((CROWS, CCHUNK), jnp.int32),
            pltpu.VMEM((CROWS, CCHUNK), jnp.int32),
            pltpu.VMEM((CCHUNK,), jnp.int32),
            pltpu.VMEM((CCHUNK,), jnp.int32),
            pltpu.VMEM((CCHUNK, 128), jnp.float32),
            pltpu.VMEM((CCHUNK, 128), jnp.float32),
            pltpu.VMEM_SHARED((AROW, 128), jnp.float32),
            pltpu.SemaphoreType.DMA,
            pltpu.SemaphoreType.DMA,
            pltpu.SemaphoreType.DMA,
            pltpu.SemaphoreType.DMA,
        ],
    )
    return f(h0, h1, src2d, dst2d, zeros2d)


def _deg_body(i0, i1, i2, i3, i4, i5, zeros640, o0, o1, o2, o3, o4, o5,
              idxblk, ones, a0, a1, a2, sem):
    s = lax.axis_index("s")
    c = lax.axis_index("c")

    for j in range(8):
        ones[pl.ds(16 * j, 16)] = jnp.ones((16,), jnp.float32)

    def histo(idx_hbm, acc):
        pltpu.sync_copy(zeros640, acc.at[pl.ds(s * 640, 640)])
        pltpu.sync_copy(idx_hbm.at[pl.ds(s * ROWS_PER_TILE, ROWS_PER_TILE)],
                        idxblk)
        plsc.subcore_barrier()

        def step(i, _):
            for j in range(10):
                pltpu.async_copy(ones.at[pl.ds(0, CHUNK)],
                                 acc.at[idxblk.at[10 * i + j]], sem, add=True)
            for j in range(10):
                pltpu.make_async_copy(ones.at[pl.ds(0, CHUNK)],
                                      acc.at[idxblk.at[10 * i + j]], sem).wait()
            return 0

        lax.fori_loop(0, ROWS_PER_TILE // 10, step, 0)
        plsc.subcore_barrier()

    def emit(idx_hbm, acc, out_hbm):
        histo(idx_hbm, acc)
        pltpu.sync_copy(acc.at[pl.ds(s * 640, 640)],
                        out_hbm.at[pl.ds(s * 640, 640)])

    @pl.when(c == 0)
    def _():
        emit(i0, a0, o0)
        emit(i1, a1, o1)
        emit(i2, a2, o2)

    @pl.when(c == 1)
    def _():
        emit(i3, a0, o3)
        emit(i4, a1, o4)
        emit(i5, a2, o5)


def _sc_degrees(idx6, zeros640):
    f = pl.kernel(
        _deg_body,
        out_type=[jax.ShapeDtypeStruct((NPAD,), jnp.float32)] * 6,
        mesh=_mesh,
        scratch_types=[
            pltpu.VMEM((ROWS_PER_TILE, CHUNK), jnp.int32),
            pltpu.VMEM((128,), jnp.float32),
            pltpu.VMEM_SHARED((NPAD,), jnp.float32),
            pltpu.VMEM_SHARED((NPAD,), jnp.float32),
            pltpu.VMEM_SHARED((NPAD,), jnp.float32),
            pltpu.SemaphoreType.DMA,
        ],
    )
    return f(*idx6, zeros640)


BR = 1000
NBLK = N // BR


def _acc_scalar(ref, i, s):
    @pl.when(i == 0)
    def _():
        ref[...] = jnp.zeros_like(ref)
    ref[...] += jnp.full(ref.shape, s, jnp.float32)


def _pre_body(x_ref, w_ref, ns_ref, a_ref, c_ref, *out_refs):
    x = x_ref[...] * a_ref[...][None, :] + c_ref[...][None, :]
    h = jnp.dot(x, w_ref[...], preferred_element_type=jnp.float32)
    for r in range(3):
        hr = h[:, 256 * r:256 * (r + 1)] * ns_ref[0, r, :][:, None]
        out_refs[2 * r][...] = hr[:, :128]
        out_refs[2 * r + 1][...] = hr[:, 128:]


def _pre_conv(x, wcat, ns3, a, c):
    din = x.shape[1]
    return pl.pallas_call(
        _pre_body,
        grid=(NBLK,),
        in_specs=[
            pl.BlockSpec((BR, din), lambda i: (i, 0)),
            pl.BlockSpec((din, 768), lambda i: (0, 0)),
            pl.BlockSpec((1, 3, BR), lambda i: (i, 0, 0)),
            pl.BlockSpec((din,), lambda i: (0,)),
            pl.BlockSpec((din,), lambda i: (0,)),
        ],
        out_specs=[pl.BlockSpec((BR, 128), lambda i: (i, 0))] * 6,
        out_shape=[jax.ShapeDtypeStruct((N, 128), jnp.float32)] * 6,
    )(x, wcat, ns3, a, c)


def _combine(acc_refs, nd_ref, bsum_ref):
    y = 0.0
    for r in range(3):
        ar = jnp.concatenate([acc_refs[2 * r][...], acc_refs[2 * r + 1][...]],
                             axis=1)
        y = y + ar * nd_ref[0, r, :][:, None]
    return y + bsum_ref[...][None, :]


def _post_body(a0, a1, a2, a3, a4, a5, nd_ref, bsum_ref, w_ref, b_ref,
               u_ref, ssum_ref, ssq_ref):
    i = pl.program_id(0)
    y = _combine((a0, a1, a2, a3, a4, a5), nd_ref, bsum_ref)
    t = jnp.dot(y, w_ref[...], preferred_element_type=jnp.float32)
    u = jax.nn.relu(t + b_ref[...][None, :])
    u_ref[...] = u

    @pl.when(i == 0)
    def _():
        ssum_ref[...] = jnp.zeros_like(ssum_ref)
        ssq_ref[...] = jnp.zeros_like(ssq_ref)
    ssum_ref[...] += jnp.sum(u, axis=0, keepdims=True)
    ssq_ref[...] += jnp.sum(u * u, axis=0, keepdims=True)


def _post_conv_bnstats(accs, nd3, bsum, w, b):
    return pl.pallas_call(
        _post_body,
        grid=(NBLK,),
        in_specs=[pl.BlockSpec((BR, 128), lambda i: (i, 0))] * 6 + [
            pl.BlockSpec((1, 3, BR), lambda i: (i, 0, 0)),
            pl.BlockSpec((D_H,), lambda i: (0,)),
            pl.BlockSpec((D_H, D_H), lambda i: (0, 0)),
            pl.BlockSpec((D_H,), lambda i: (0,)),
        ],
        out_specs=[
            pl.BlockSpec((BR, D_H), lambda i: (i, 0)),
            pl.BlockSpec((1, D_H), lambda i: (0, 0)),
            pl.BlockSpec((1, D_H), lambda i: (0, 0)),
        ],
        out_shape=[
            jax.ShapeDtypeStruct((N, D_H), jnp.float32),
            jax.ShapeDtypeStruct((1, D_H), jnp.float32),
            jax.ShapeDtypeStruct((1, D_H), jnp.float32),
        ],
    )(*accs, nd3, bsum, w, b)


def _final_body(a0, a1, a2, a3, a4, a5, nd_ref, bsum_ref, w_ref, b_ref,
                xin_ref, mi_ref, loss_ref, *, masked):
    i = pl.program_id(0)
    y = _combine((a0, a1, a2, a3, a4, a5), nd_ref, bsum_ref)
    xr = jnp.dot(y, w_ref[...], preferred_element_type=jnp.float32)
    xr = xr + b_ref[...][None, :]
    xin = xin_ref[...]
    if masked:
        na = jnp.sqrt(jnp.sum(xr * xr, axis=1, keepdims=True))
        a = xr / jnp.maximum(na, 1e-12)
        nb = jnp.sqrt(jnp.sum(xin * xin, axis=1, keepdims=True))
        bv = xin / jnp.maximum(nb, 1e-12)
        pn = (1.0 - jnp.sum(a * bv, axis=1)) ** SCE
        s = jnp.sum(pn * mi_ref[0, 0, :])
    else:
        s = jnp.sum((xr - xin) ** 2)
    _acc_scalar(loss_ref, i, s)


def _final_conv(accs, nd3, bsum, w, b, x_in, mi3d, masked):
    return pl.pallas_call(
        functools.partial(_final_body, masked=masked),
        grid=(NBLK,),
        in_specs=[pl.BlockSpec((BR, 128), lambda i: (i, 0))] * 6 + [
            pl.BlockSpec((1, 3, BR), lambda i: (i, 0, 0)),
            pl.BlockSpec((D_H,), lambda i: (0,)),
            pl.BlockSpec((D_H, D_IN), lambda i: (0, 0)),
            pl.BlockSpec((D_IN,), lambda i: (0,)),
            pl.BlockSpec((BR, D_IN), lambda i: (i, 0)),
            pl.BlockSpec((1, 1, BR), lambda i: (i, 0, 0)),
        ],
        out_specs=pl.BlockSpec((1, 128), lambda i: (0, 0)),
        out_shape=jax.ShapeDtypeStruct((1, 128), jnp.float32),
    )(*accs, nd3, bsum, w, b, x_in, mi3d)


def _vq_body(xn_ref, cn_ref, mk_ref, e_ref, em_ref, mi_ref, qp_ref, mip_ref):
    i = pl.program_id(0)
    xn = xn_ref[...]
    cn = cn_ref[...]
    dot = lax.dot_general(xn, cn, (((1,), (1,)), ((), ())),
                          preferred_element_type=jnp.float32)
    a = jnp.sum(xn * xn, axis=1, keepdims=True)
    sc = jnp.sum(cn * cn, axis=1)[None, :]
    d = a + sc - 2.0 * dot
    idx = jnp.argmin(d, axis=1).astype(jnp.int32)
    oh = (idx[:, None] == lax.broadcasted_iota(jnp.int32, (1, K), 1))
    ohf = oh.astype(jnp.float32)
    quant = jnp.dot(ohf, cn, preferred_element_type=jnp.float32)
    miv = jnp.dot(ohf, mk_ref[...], preferred_element_type=jnp.float32)
    e = xn + (quant - xn)
    e_ref[...] = e
    em_ref[...] = e * (1.0 - miv)[:, None]
    mi_ref[0, 0, :] = miv
    _acc_scalar(qp_ref, i, jnp.sum((quant - xn) ** 2))
    _acc_scalar(mip_ref, i, jnp.sum(miv))


def _vq_block(xn, cn, maskf):
    return pl.pallas_call(
        _vq_body,
        grid=(NBLK,),
        in_specs=[
            pl.BlockSpec((BR, D_H), lambda i: (i, 0)),
            pl.BlockSpec((K, D_H), lambda i: (0, 0)),
            pl.BlockSpec((K,), lambda i: (0,)),
        ],
        out_specs=[
            pl.BlockSpec((BR, D_H), lambda i: (i, 0)),
            pl.BlockSpec((BR, D_H), lambda i: (i, 0)),
            pl.BlockSpec((1, 1, BR), lambda i: (i, 0, 0)),
            pl.BlockSpec((1, 128), lambda i: (0, 0)),
            pl.BlockSpec((1, 128), lambda i: (0, 0)),
        ],
        out_shape=[
            jax.ShapeDtypeStruct((N, D_H), jnp.float32),
            jax.ShapeDtypeStruct((N, D_H), jnp.float32),
            jax.ShapeDtypeStruct((NBLK, 1, BR), jnp.float32),
            jax.ShapeDtypeStruct((1, 128), jnp.float32),
            jax.ShapeDtypeStruct((1, 128), jnp.float32),
        ],
    )(xn, cn, maskf)


def _normalize(x, eps=1e-12):
    n = jnp.linalg.norm(x, axis=-1, keepdims=True)
    return x / jnp.maximum(n, eps)


def _norm_coeff(deg):
    return jnp.where(deg > 0, 1.0 / jnp.sqrt(jnp.maximum(deg, 1e-9)), 0.0)


def _bn(x, g, b):
    mu = jnp.mean(x, axis=0)
    var = jnp.var(x, axis=0)
    return (x - mu) / jnp.sqrt(var + 1e-5) * g + b


def _hetero_exact(x, conv_params, ed):
    out = 0.0
    for r in ('SEQ', 'KNN', 'DIS'):
        W, b = conv_params[r]['W'], conv_params[r]['b']
        src, dst, ns, nd = ed[r][2], ed[r][3], ed[r][4], ed[r][5]
        h = x @ W
        m = h[src] * ns[src][:, None]
        agg = jnp.zeros((N, h.shape[1]), h.dtype).at[dst].add(m)
        out = out + (agg * nd[:, None] + b)
    return out


def _encode(x, enc, ed):
    for l in range(2):
        x = _hetero_exact(x, enc['convs'][l], ed)
        x = x @ enc['fcs'][l]['W'] + enc['fcs'][l]['b']
        x = _bn(jax.nn.relu(x), enc['bns'][l]['g'], enc['bns'][l]['b'])
    return x


_RELS = ('SEQ', 'KNN', 'DIS')


def _hetero_sc_accs(h6, ed, zeros2d):
    accs = []
    for r_i, r in enumerate(_RELS):
        src2d, dst2d = ed[r][0], ed[r][1]
        a0, a1 = _sc_conv(h6[2 * r_i], h6[2 * r_i + 1], src2d, dst2d, zeros2d)
        accs += [a0, a1]
    return accs


def _decode_pallas(v, dec, ed, ns3, nd3, zeros2d, x_in, mi3d, masked):
    wcat0 = jnp.concatenate([dec['convs'][0][r]['W'] for r in _RELS], axis=1)
    bsum0 = sum(dec['convs'][0][r]['b'] for r in _RELS)
    wcat1 = jnp.concatenate([dec['convs'][1][r]['W'] for r in _RELS], axis=1)
    bsum1 = sum(dec['convs'][1][r]['b'] for r in _RELS)

    one = jnp.ones((D_H,), jnp.float32)
    zero = jnp.zeros((D_H,), jnp.float32)
    h6 = _pre_conv(v, wcat0, ns3, one, zero)
    accs = _hetero_sc_accs(h6, ed, zeros2d)
    u, ssum, ssq = _post_conv_bnstats(accs, nd3, bsum0,
                                      dec['fcs'][0]['W'], dec['fcs'][0]['b'])
    mu = ssum[0] / N
    var = ssq[0] / N - mu * mu
    abn = dec['bns'][0]['g'] / jnp.sqrt(var + 1e-5)
    cbn = dec['bns'][0]['b'] - mu * abn
    h6b = _pre_conv(u, wcat1, ns3, abn, cbn)
    accs2 = _hetero_sc_accs(h6b, ed, zeros2d)
    loss = _final_conv(accs2, nd3, bsum1, dec['fcs'][1]['W'],
                       dec['fcs'][1]['b'], x_in, mi3d, masked)
    return loss[0, 0]


def kernel(x, edge_index_seq, edge_index_knn, edge_index_dis, mask, params,
           codebook):
    maskf = mask.astype(jnp.float32)
    zeros2d = jnp.zeros((AROW // 16, 128), jnp.float32)
    zeros640 = jnp.zeros((640,), jnp.float32)
    pad_src = jnp.arange(EPAD - E, dtype=jnp.int32) % N
    pad_dst = jnp.full((EPAD - E,), 1 << 30, jnp.int32)

    e2d, ec = {}, {}
    for r, ei in (('SEQ', edge_index_seq), ('KNN', edge_index_knn),
                  ('DIS', edge_index_dis)):
        e2d[r] = (ei[0].reshape(NROW2D, CHUNK), ei[1].reshape(NROW2D, CHUNK))
        ec[r] = (jnp.concatenate([ei[0], pad_src]).reshape(EPAD // CCHUNK, CCHUNK),
                 jnp.concatenate([ei[1], pad_dst]).reshape(EPAD // CCHUNK, CCHUNK))

    idx6 = [e2d['SEQ'][0], e2d['SEQ'][1], e2d['KNN'][0],
            e2d['KNN'][1], e2d['DIS'][0], e2d['DIS'][1]]
    degs = _sc_degrees(idx6, zeros640)
    srcdst = {'SEQ': edge_index_seq, 'KNN': edge_index_knn,
              'DIS': edge_index_dis}
    ed = {}
    for k, r in enumerate(('SEQ', 'KNN', 'DIS')):
        ns = _norm_coeff(degs[2 * k][:N])
        nd = _norm_coeff(degs[2 * k + 1][:N])
        ed[r] = (ec[r][0], ec[r][1], srcdst[r][0], srcdst[r][1], ns, nd)

    ns3 = jnp.stack([ed[r][4] for r in _RELS]).reshape(3, NBLK, BR).transpose(1, 0, 2)
    nd3 = jnp.stack([ed[r][5] for r in _RELS]).reshape(3, NBLK, BR).transpose(1, 0, 2)

    x_in = x
    z = _encode(x_in, params['enc'], ed)
    xn = _normalize(z)
    cn = _normalize(codebook)
    e, e_masked, mi3d, qp, mip = _vq_block(xn, cn, maskf)
    q_loss = qp[0, 0] / (N * D_H)
    e_q_loss = q_loss + CC * q_loss
    rp = _decode_pallas(e, params['dec'], ed, ns3, nd3, zeros2d,
                        x_in, mi3d, masked=False)
    recon_loss = rp / (N * D_IN)
    mp = _decode_pallas(e_masked, params['dec'], ed, ns3, nd3, zeros2d,
                        x_in, mi3d, masked=True)
    mask_loss = mp / (mip[0, 0] + 1e-12)
    return z, e_masked, e_q_loss, recon_loss, mask_loss


# speedup vs baseline: 1.3747x; 1.3173x over previous
import functools
import jax
import jax.numpy as jnp
from jax import lax
from jax.experimental import pallas as pl
from jax.experimental.pallas import tpu as pltpu
from jax.experimental.pallas import tpu_sc as plsc

N = 10000
E = 160000
D_IN = 128
D_H = 256
K = 512
CC = 0.25
SCE = 2

NTILE = 16
CHUNK = 125
ROWS_PER_TILE = 80
NROW2D = E // CHUNK
NPAD = 10240
NPT = NPAD // NTILE

_mesh = plsc.VectorSubcoreMesh(core_axis_name="c", subcore_axis_name="s")


HROW = 5120
AROW = 5248
JUNK = 5192
CCHUNK = 128
CROWS = 80
EPAD = 16 * CROWS * CCHUNK


def _remap(idx_d, g, p, idx_r):
    for j in range(CCHUNK // 16):
        v = idx_d[g, pl.ds(16 * j, 16)]
        lo = v - jnp.int32(p * HROW)
        ok = (lo >= 0) & (lo < HROW)
        idx_r[pl.ds(16 * j, 16)] = jnp.where(ok, lo, jnp.int32(JUNK))


def _conv_body(h0, h1, src2d, dst2d, zeros2d, out0, out1,
               idx_s, idx_d, idx_r0, idx_r1, rows0, rows1, acc,
               sem0, sem1, sem2, sem3):
    s = lax.axis_index("s")
    c = lax.axis_index("c")

    pltpu.sync_copy(src2d.at[pl.ds(s * CROWS, CROWS)], idx_s)
    pltpu.sync_copy(dst2d.at[pl.ds(s * CROWS, CROWS)], idx_d)

    def row_pass(p, h_hbm, out_hbm):
        pltpu.sync_copy(zeros2d, acc.at[pl.ds(s * (AROW // 16), AROW // 16)])
        plsc.subcore_barrier()

        pltpu.async_copy(h_hbm.at[idx_s.at[0]], rows0, sem0)
        pltpu.async_copy(h_hbm.at[idx_s.at[1]], rows1, sem1)

        def step(i, _):
            g = 2 * i
            pltpu.make_async_copy(h_hbm.at[idx_s.at[g]], rows0, sem0).wait()
            _remap(idx_d, g, p, idx_r0)
            pltpu.sync_copy(rows0, acc.at[idx_r0], add=True)

            @pl.when(g + 2 < CROWS)
            def _():
                pltpu.async_copy(h_hbm.at[idx_s.at[g + 2]], rows0, sem0)

            pltpu.make_async_copy(h_hbm.at[idx_s.at[g + 1]], rows1, sem1).wait()
            _remap(idx_d, g + 1, p, idx_r1)
            pltpu.sync_copy(rows1, acc.at[idx_r1], add=True)

            @pl.when(g + 3 < CROWS)
            def _():
                pltpu.async_copy(h_hbm.at[idx_s.at[g + 3]], rows1, sem1)
            return 0

        lax.fori_loop(0, CROWS // 2, step, 0)
        plsc.subcore_barrier()
        pltpu.sync_copy(acc.at[pl.ds(s * 320, 320)],
                        out_hbm.at[pl.ds(p * HROW + s * 320, 320)])
        plsc.subcore_barrier()

    def col_half(h_hbm, out_hbm):
        row_pass(0, h_hbm, out_hbm)
        row_pass(1, h_hbm, out_hbm)

    @pl.when(c == 0)
    def _():
        col_half(h0, out0)

    @pl.when(c == 1)
    def _():
        col_half(h1, out1)


def _sc_conv(h0, h1, src2d, dst2d, zeros2d):
    f = pl.kernel(
        _conv_body,
        out_type=[jax.ShapeDtypeStruct((NPAD, 128), jnp.float32)] * 2,
        mesh=_mesh,
        scratch_types=[
            pltpu.VMEM((CROWS, CCHUNK), jnp.int32),
            pltpu.VMEM((CROWS, CCHUNK), jnp.int32),
            pltpu.VMEM((CCHUNK,), jnp.int32),
            pltpu.VMEM((CCHUNK,), jnp.int32),
            pltpu.VMEM((CCHUNK, 128), jnp.float32),
            pltpu.VMEM((CCHUNK, 128), jnp.float32),
            pltpu.VMEM_SHARED((AROW, 128), jnp.float32),
            pltpu.SemaphoreType.DMA,
            pltpu.SemaphoreType.DMA,
            pltpu.SemaphoreType.DMA,
            pltpu.SemaphoreType.DMA,
        ],
    )
    return f(h0, h1, src2d, dst2d, zeros2d)


GCHUNKS = EPAD // CCHUNK // 32


def _gather_body(hs, srcp, mp, idx_g, grow0, grow1, gsem0, gsem1):
    s = lax.axis_index("s")
    c = lax.axis_index("c")
    base = (c * NTILE + s) * GCHUNKS
    pltpu.sync_copy(srcp.at[pl.ds(base, GCHUNKS)], idx_g)

    pltpu.async_copy(hs.at[idx_g.at[0]], grow0, gsem0)
    pltpu.async_copy(hs.at[idx_g.at[1]], grow1, gsem1)

    def step(i, _):
        g = 2 * i
        pltpu.make_async_copy(hs.at[idx_g.at[g]], grow0, gsem0).wait()
        pltpu.sync_copy(grow0, mp.at[pl.ds((base + g) * CCHUNK, CCHUNK)])

        @pl.when(g + 2 < GCHUNKS)
        def _():
            pltpu.async_copy(hs.at[idx_g.at[g + 2]], grow0, gsem0)

        pltpu.make_async_copy(hs.at[idx_g.at[g + 1]], grow1, gsem1).wait()
        pltpu.sync_copy(grow1, mp.at[pl.ds((base + g + 1) * CCHUNK, CCHUNK)])

        @pl.when(g + 3 < GCHUNKS)
        def _():
            pltpu.async_copy(hs.at[idx_g.at[g + 3]], grow1, gsem1)
        return 0

    lax.fori_loop(0, GCHUNKS // 2, step, 0)


def _sc_gather(hs, srcp):
    f = pl.kernel(
        _gather_body,
        out_type=jax.ShapeDtypeStruct((EPAD, D_H), jnp.float32),
        mesh=_mesh,
        scratch_types=[
            pltpu.VMEM((GCHUNKS, CCHUNK), jnp.int32),
            pltpu.VMEM((CCHUNK, D_H), jnp.float32),
            pltpu.VMEM((CCHUNK, D_H), jnp.float32),
            pltpu.SemaphoreType.DMA,
            pltpu.SemaphoreType.DMA,
        ],
    )
    return f(hs, srcp)


def _deg_body(i0, i1, i2, i3, i4, i5, zeros640, o0, o1, o2, o3, o4, o5,
              idxblk, ones, a0, a1, a2, sem):
    s = lax.axis_index("s")
    c = lax.axis_index("c")

    for j in range(8):
        ones[pl.ds(16 * j, 16)] = jnp.ones((16,), jnp.float32)

    def histo(idx_hbm, acc):
        pltpu.sync_copy(zeros640, acc.at[pl.ds(s * 640, 640)])
        pltpu.sync_copy(idx_hbm.at[pl.ds(s * ROWS_PER_TILE, ROWS_PER_TILE)],
                        idxblk)
        plsc.subcore_barrier()

        def step(i, _):
            for j in range(10):
                pltpu.async_copy(ones.at[pl.ds(0, CHUNK)],
                                 acc.at[idxblk.at[10 * i + j]], sem, add=True)
            for j in range(10):
                pltpu.make_async_copy(ones.at[pl.ds(0, CHUNK)],
                                      acc.at[idxblk.at[10 * i + j]], sem).wait()
            return 0

        lax.fori_loop(0, ROWS_PER_TILE // 10, step, 0)
        plsc.subcore_barrier()

    def emit(idx_hbm, acc, out_hbm):
        histo(idx_hbm, acc)
        pltpu.sync_copy(acc.at[pl.ds(s * 640, 640)],
                        out_hbm.at[pl.ds(s * 640, 640)])

    @pl.when(c == 0)
    def _():
        emit(i0, a0, o0)
        emit(i1, a1, o1)
        emit(i2, a2, o2)

    @pl.when(c == 1)
    def _():
        emit(i3, a0, o3)
        emit(i4, a1, o4)
        emit(i5, a2, o5)


def _sc_degrees(idx6, zeros640):
    f = pl.kernel(
        _deg_body,
        out_type=[jax.ShapeDtypeStruct((NPAD,), jnp.float32)] * 6,
        mesh=_mesh,
        scratch_types=[
            pltpu.VMEM((ROWS_PER_TILE, CHUNK), jnp.int32),
            pltpu.VMEM((128,), jnp.float32),
            pltpu.VMEM_SHARED((NPAD,), jnp.float32),
            pltpu.VMEM_SHARED((NPAD,), jnp.float32),
            pltpu.VMEM_SHARED((NPAD,), jnp.float32),
            pltpu.SemaphoreType.DMA,
        ],
    )
    return f(*idx6, zeros640)


BR = 1000
NBLK = N // BR


def _acc_scalar(ref, i, s):
    @pl.when(i == 0)
    def _():
        ref[...] = jnp.zeros_like(ref)
    ref[...] += jnp.full(ref.shape, s, jnp.float32)


def _pre_body(x_ref, w_ref, ns_ref, a_ref, c_ref, *out_refs):
    x = x_ref[...] * a_ref[...][None, :] + c_ref[...][None, :]
    h = jnp.dot(x, w_ref[...], preferred_element_type=jnp.float32)
    for r in range(3):
        hr = h[:, 256 * r:256 * (r + 1)] * ns_ref[0, r, :][:, None]
        out_refs[2 * r][...] = hr[:, :128]
        out_refs[2 * r + 1][...] = hr[:, 128:]


def _pre_conv(x, wcat, ns3, a, c):
    din = x.shape[1]
    return pl.pallas_call(
        _pre_body,
        grid=(NBLK,),
        in_specs=[
            pl.BlockSpec((BR, din), lambda i: (i, 0)),
            pl.BlockSpec((din, 768), lambda i: (0, 0)),
            pl.BlockSpec((1, 3, BR), lambda i: (i, 0, 0)),
            pl.BlockSpec((din,), lambda i: (0,)),
            pl.BlockSpec((din,), lambda i: (0,)),
        ],
        out_specs=[pl.BlockSpec((BR, 128), lambda i: (i, 0))] * 6,
        out_shape=[jax.ShapeDtypeStruct((N, 128), jnp.float32)] * 6,
    )(x, wcat, ns3, a, c)


def _combine(acc_refs, nd_ref, bsum_ref):
    y = 0.0
    for r in range(3):
        ar = jnp.concatenate([acc_refs[2 * r][...], acc_refs[2 * r + 1][...]],
                             axis=1)
        y = y + ar * nd_ref[0, r, :][:, None]
    return y + bsum_ref[...][None, :]


def _post_body(a0, a1, a2, a3, a4, a5, nd_ref, bsum_ref, w_ref, b_ref,
               u_ref, ssum_ref, ssq_ref):
    i = pl.program_id(0)
    y = _combine((a0, a1, a2, a3, a4, a5), nd_ref, bsum_ref)
    t = jnp.dot(y, w_ref[...], preferred_element_type=jnp.float32)
    u = jax.nn.relu(t + b_ref[...][None, :])
    u_ref[...] = u

    @pl.when(i == 0)
    def _():
        ssum_ref[...] = jnp.zeros_like(ssum_ref)
        ssq_ref[...] = jnp.zeros_like(ssq_ref)
    ssum_ref[...] += jnp.sum(u, axis=0, keepdims=True)
    ssq_ref[...] += jnp.sum(u * u, axis=0, keepdims=True)


def _post_conv_bnstats(accs, nd3, bsum, w, b):
    return pl.pallas_call(
        _post_body,
        grid=(NBLK,),
        in_specs=[pl.BlockSpec((BR, 128), lambda i: (i, 0))] * 6 + [
            pl.BlockSpec((1, 3, BR), lambda i: (i, 0, 0)),
            pl.BlockSpec((D_H,), lambda i: (0,)),
            pl.BlockSpec((D_H, D_H), lambda i: (0, 0)),
            pl.BlockSpec((D_H,), lambda i: (0,)),
        ],
        out_specs=[
            pl.BlockSpec((BR, D_H), lambda i: (i, 0)),
            pl.BlockSpec((1, D_H), lambda i: (0, 0)),
            pl.BlockSpec((1, D_H), lambda i: (0, 0)),
        ],
        out_shape=[
            jax.ShapeDtypeStruct((N, D_H), jnp.float32),
            jax.ShapeDtypeStruct((1, D_H), jnp.float32),
            jax.ShapeDtypeStruct((1, D_H), jnp.float32),
        ],
    )(*accs, nd3, bsum, w, b)


def _final_body(a0, a1, a2, a3, a4, a5, nd_ref, bsum_ref, w_ref, b_ref,
                xin_ref, mi_ref, loss_ref, *, masked):
    i = pl.program_id(0)
    y = _combine((a0, a1, a2, a3, a4, a5), nd_ref, bsum_ref)
    xr = jnp.dot(y, w_ref[...], preferred_element_type=jnp.float32)
    xr = xr + b_ref[...][None, :]
    xin = xin_ref[...]
    if masked:
        na = jnp.sqrt(jnp.sum(xr * xr, axis=1, keepdims=True))
        a = xr / jnp.maximum(na, 1e-12)
        nb = jnp.sqrt(jnp.sum(xin * xin, axis=1, keepdims=True))
        bv = xin / jnp.maximum(nb, 1e-12)
        pn = (1.0 - jnp.sum(a * bv, axis=1)) ** SCE
        s = jnp.sum(pn * mi_ref[0, 0, :])
    else:
        s = jnp.sum((xr - xin) ** 2)
    _acc_scalar(loss_ref, i, s)


def _final_conv(accs, nd3, bsum, w, b, x_in, mi3d, masked):
    return pl.pallas_call(
        functools.partial(_final_body, masked=masked),
        grid=(NBLK,),
        in_specs=[pl.BlockSpec((BR, 128), lambda i: (i, 0))] * 6 + [
            pl.BlockSpec((1, 3, BR), lambda i: (i, 0, 0)),
            pl.BlockSpec((D_H,), lambda i: (0,)),
            pl.BlockSpec((D_H, D_IN), lambda i: (0, 0)),
            pl.BlockSpec((D_IN,), lambda i: (0,)),
            pl.BlockSpec((BR, D_IN), lambda i: (i, 0)),
            pl.BlockSpec((1, 1, BR), lambda i: (i, 0, 0)),
        ],
        out_specs=pl.BlockSpec((1, 128), lambda i: (0, 0)),
        out_shape=jax.ShapeDtypeStruct((1, 128), jnp.float32),
    )(*accs, nd3, bsum, w, b, x_in, mi3d)


def _vq_body(xn_ref, cn_ref, mk_ref, e_ref, em_ref, mi_ref, qp_ref, mip_ref):
    i = pl.program_id(0)
    xn = xn_ref[...]
    cn = cn_ref[...]
    dot = lax.dot_general(xn, cn, (((1,), (1,)), ((), ())),
                          preferred_element_type=jnp.float32)
    a = jnp.sum(xn * xn, axis=1, keepdims=True)
    sc = jnp.sum(cn * cn, axis=1)[None, :]
    d = a + sc - 2.0 * dot
    idx = jnp.argmin(d, axis=1).astype(jnp.int32)
    oh = (idx[:, None] == lax.broadcasted_iota(jnp.int32, (1, K), 1))
    ohf = oh.astype(jnp.float32)
    quant = jnp.dot(ohf, cn, preferred_element_type=jnp.float32)
    miv = jnp.dot(ohf, mk_ref[...], preferred_element_type=jnp.float32)
    e = xn + (quant - xn)
    e_ref[...] = e
    em_ref[...] = e * (1.0 - miv)[:, None]
    mi_ref[0, 0, :] = miv
    _acc_scalar(qp_ref, i, jnp.sum((quant - xn) ** 2))
    _acc_scalar(mip_ref, i, jnp.sum(miv))


def _vq_block(xn, cn, maskf):
    return pl.pallas_call(
        _vq_body,
        grid=(NBLK,),
        in_specs=[
            pl.BlockSpec((BR, D_H), lambda i: (i, 0)),
            pl.BlockSpec((K, D_H), lambda i: (0, 0)),
            pl.BlockSpec((K,), lambda i: (0,)),
        ],
        out_specs=[
            pl.BlockSpec((BR, D_H), lambda i: (i, 0)),
            pl.BlockSpec((BR, D_H), lambda i: (i, 0)),
            pl.BlockSpec((1, 1, BR), lambda i: (i, 0, 0)),
            pl.BlockSpec((1, 128), lambda i: (0, 0)),
            pl.BlockSpec((1, 128), lambda i: (0, 0)),
        ],
        out_shape=[
            jax.ShapeDtypeStruct((N, D_H), jnp.float32),
            jax.ShapeDtypeStruct((N, D_H), jnp.float32),
            jax.ShapeDtypeStruct((NBLK, 1, BR), jnp.float32),
            jax.ShapeDtypeStruct((1, 128), jnp.float32),
            jax.ShapeDtypeStruct((1, 128), jnp.float32),
        ],
    )(xn, cn, maskf)


def _normalize(x, eps=1e-12):
    n = jnp.linalg.norm(x, axis=-1, keepdims=True)
    return x / jnp.maximum(n, eps)


def _norm_coeff(deg):
    return jnp.where(deg > 0, 1.0 / jnp.sqrt(jnp.maximum(deg, 1e-9)), 0.0)


def _bn(x, g, b):
    mu = jnp.mean(x, axis=0)
    var = jnp.var(x, axis=0)
    return (x - mu) / jnp.sqrt(var + 1e-5) * g + b


def _hetero_exact(x, conv_params, ed):
    out = 0.0
    for r in ('SEQ', 'KNN', 'DIS'):
        W, b = conv_params[r]['W'], conv_params[r]['b']
        srcp, dst, ns, nd = ed[r][0], ed[r][3], ed[r][4], ed[r][5]
        h = x @ W
        hs = h * ns[:, None]
        m = _sc_gather(hs, srcp)[:E]
        agg = jnp.zeros((N, h.shape[1]), h.dtype).at[dst].add(m)
        out = out + (agg * nd[:, None] + b)
    return out


def _encode(x, enc, ed):
    for l in range(2):
        x = _hetero_exact(x, enc['convs'][l], ed)
        x = x @ enc['fcs'][l]['W'] + enc['fcs'][l]['b']
        x = _bn(jax.nn.relu(x), enc['bns'][l]['g'], enc['bns'][l]['b'])
    return x


_RELS = ('SEQ', 'KNN', 'DIS')


def _hetero_sc_accs(h6, ed, zeros2d):
    accs = []
    for r_i, r in enumerate(_RELS):
        src2d, dst2d = ed[r][0], ed[r][1]
        a0, a1 = _sc_conv(h6[2 * r_i], h6[2 * r_i + 1], src2d, dst2d, zeros2d)
        accs += [a0, a1]
    return accs


def _decode_pallas(v, dec, ed, ns3, nd3, zeros2d, x_in, mi3d, masked):
    wcat0 = jnp.concatenate([dec['convs'][0][r]['W'] for r in _RELS], axis=1)
    bsum0 = sum(dec['convs'][0][r]['b'] for r in _RELS)
    wcat1 = jnp.concatenate([dec['convs'][1][r]['W'] for r in _RELS], axis=1)
    bsum1 = sum(dec['convs'][1][r]['b'] for r in _RELS)

    one = jnp.ones((D_H,), jnp.float32)
    zero = jnp.zeros((D_H,), jnp.float32)
    h6 = _pre_conv(v, wcat0, ns3, one, zero)
    accs = _hetero_sc_accs(h6, ed, zeros2d)
    u, ssum, ssq = _post_conv_bnstats(accs, nd3, bsum0,
                                      dec['fcs'][0]['W'], dec['fcs'][0]['b'])
    mu = ssum[0] / N
    var = ssq[0] / N - mu * mu
    abn = dec['bns'][0]['g'] / jnp.sqrt(var + 1e-5)
    cbn = dec['bns'][0]['b'] - mu * abn
    h6b = _pre_conv(u, wcat1, ns3, abn, cbn)
    accs2 = _hetero_sc_accs(h6b, ed, zeros2d)
    loss = _final_conv(accs2, nd3, bsum1, dec['fcs'][1]['W'],
                       dec['fcs'][1]['b'], x_in, mi3d, masked)
    return loss[0, 0]


def kernel(x, edge_index_seq, edge_index_knn, edge_index_dis, mask, params,
           codebook):
    maskf = mask.astype(jnp.float32)
    zeros2d = jnp.zeros((AROW // 16, 128), jnp.float32)
    zeros640 = jnp.zeros((640,), jnp.float32)
    pad_src = jnp.arange(EPAD - E, dtype=jnp.int32) % N
    pad_dst = jnp.full((EPAD - E,), 1 << 30, jnp.int32)

    e2d, ec = {}, {}
    for r, ei in (('SEQ', edge_index_seq), ('KNN', edge_index_knn),
                  ('DIS', edge_index_dis)):
        e2d[r] = (ei[0].reshape(NROW2D, CHUNK), ei[1].reshape(NROW2D, CHUNK))
        ec[r] = (jnp.concatenate([ei[0], pad_src]).reshape(EPAD // CCHUNK, CCHUNK),
                 jnp.concatenate([ei[1], pad_dst]).reshape(EPAD // CCHUNK, CCHUNK))

    idx6 = [e2d['SEQ'][0], e2d['SEQ'][1], e2d['KNN'][0],
            e2d['KNN'][1], e2d['DIS'][0], e2d['DIS'][1]]
    degs = _sc_degrees(idx6, zeros640)
    srcdst = {'SEQ': edge_index_seq, 'KNN': edge_index_knn,
              'DIS': edge_index_dis}
    ed = {}
    for k, r in enumerate(('SEQ', 'KNN', 'DIS')):
        ns = _norm_coeff(degs[2 * k][:N])
        nd = _norm_coeff(degs[2 * k + 1][:N])
        ed[r] = (ec[r][0], ec[r][1], srcdst[r][0], srcdst[r][1], ns, nd)

    ns3 = jnp.stack([ed[r][4] for r in _RELS]).reshape(3, NBLK, BR).transpose(1, 0, 2)
    nd3 = jnp.stack([ed[r][5] for r in _RELS]).reshape(3, NBLK, BR).transpose(1, 0, 2)

    x_in = x
    z = _encode(x_in, params['enc'], ed)
    xn = _normalize(z)
    cn = _normalize(codebook)
    e, e_masked, mi3d, qp, mip = _vq_block(xn, cn, maskf)
    q_loss = qp[0, 0] / (N * D_H)
    e_q_loss = q_loss + CC * q_loss
    rp = _decode_pallas(e, params['dec'], ed, ns3, nd3, zeros2d,
                        x_in, mi3d, masked=False)
    recon_loss = rp / (N * D_IN)
    mp = _decode_pallas(e_masked, params['dec'], ed, ns3, nd3, zeros2d,
                        x_in, mi3d, masked=True)
    mask_loss = mp / (mip[0, 0] + 1e-12)
    return z, e_masked, e_q_loss, recon_loss, mask_loss
